# Initial kernel scaffold; baseline (speedup 1.0000x reference)
import jax
import jax.numpy as jnp
from jax.experimental import pallas as pl


def kernel(x, edge_index, eps0, W1_0, b1_0, W2_0, b2_0, eps1, W1_1, b1_1, W2_1, b2_1):
    raise NotImplementedError("write your pallas kernel here")


# speedup vs baseline: 6.3668x; 6.3668x over previous
import functools

import jax
import jax.numpy as jnp
from jax import lax
from jax.experimental import pallas as pl
from jax.experimental.pallas import tpu as pltpu
from jax.experimental.pallas import tpu_sc as plsc

N_NODES = 10000
N_EDGES = 320000
D = 128

CHUNK = 128
N_CHUNKS = N_EDGES // CHUNK
NC = 2
NS = 16
NW = NC * NS
MAX_CHUNKS_PER_WORKER = (N_CHUNKS + NW - 1) // NW
N_PAD = 10240
ROWS_PER_TILE = N_PAD // NS


@functools.partial(
    pl.kernel,
    out_type=jax.ShapeDtypeStruct((NC, N_PAD, D), jnp.float32),
    mesh=plsc.VectorSubcoreMesh(core_axis_name="c", subcore_axis_name="s"),
    scratch_types=[
        pltpu.VMEM((CHUNK,), jnp.int32),
        pltpu.VMEM((CHUNK,), jnp.int32),
        pltpu.VMEM((CHUNK, D), jnp.float32),
        pltpu.VMEM_SHARED((N_PAD, D), jnp.float32),
        pltpu.SemaphoreType.DMA,
    ],
)
def _sc_aggregate(src_hbm, dst_hbm, h_hbm, zeros_hbm, out_hbm,
                  src_v, dst_v, rows_v, acc_sh, sem):
    cid = lax.axis_index("c")
    sid = lax.axis_index("s")
    wid = sid * NC + cid

    pltpu.sync_copy(zeros_hbm, acc_sh.at[pl.ds(sid * ROWS_PER_TILE, ROWS_PER_TILE)])
    plsc.subcore_barrier()

    def body(i, carry):
        c = wid + i * NW

        @pl.when(c < N_CHUNKS)
        def _():
            base = c * CHUNK
            pltpu.sync_copy(src_hbm.at[pl.ds(base, CHUNK)], src_v)
            pltpu.sync_copy(dst_hbm.at[pl.ds(base, CHUNK)], dst_v)
            pltpu.async_copy(h_hbm.at[src_v], rows_v, sem).wait()
            pltpu.sync_copy(rows_v, acc_sh.at[dst_v], add=True)

        return carry

    lax.fori_loop(0, MAX_CHUNKS_PER_WORKER, body, 0)
    plsc.subcore_barrier()

    pltpu.sync_copy(
        acc_sh.at[pl.ds(sid * ROWS_PER_TILE, ROWS_PER_TILE)],
        out_hbm.at[cid, pl.ds(sid * ROWS_PER_TILE, ROWS_PER_TILE)],
    )


BLK = 2000


def _mlp_body(scale_ref, h_ref, p_ref, w1_ref, b1_ref, w2_ref, b2_ref, o_ref):
    scale = scale_ref[0]
    z = h_ref[...] * scale + p_ref[0] + p_ref[1]
    z = jnp.dot(z, w1_ref[...], preferred_element_type=jnp.float32) + b1_ref[...]
    z = jnp.maximum(z, 0.0)
    z = jnp.dot(z, w2_ref[...], preferred_element_type=jnp.float32) + b2_ref[...]
    o_ref[...] = jnp.maximum(z, 0.0)


_tc_mlp = pl.pallas_call(
    _mlp_body,
    grid=(N_NODES // BLK,),
    in_specs=[
        pl.BlockSpec(memory_space=pltpu.SMEM),
        pl.BlockSpec((BLK, D), lambda i: (i, 0)),
        pl.BlockSpec((NC, BLK, D), lambda i: (0, i, 0)),
        pl.BlockSpec((D, D), lambda i: (0, 0)),
        pl.BlockSpec((1, D), lambda i: (0, 0)),
        pl.BlockSpec((D, D), lambda i: (0, 0)),
        pl.BlockSpec((1, D), lambda i: (0, 0)),
    ],
    out_specs=pl.BlockSpec((BLK, D), lambda i: (i, 0)),
    out_shape=jax.ShapeDtypeStruct((N_NODES, D), jnp.float32),
)


def _gin_layer(h, src, dst, zeros, eps, W1, b1, W2, b2):
    parts = _sc_aggregate(src, dst, h, zeros)
    scale = (1.0 + eps).reshape((1,)).astype(jnp.float32)
    return _tc_mlp(scale, h, parts, W1, b1.reshape(1, D), W2, b2.reshape(1, D))


def kernel(x, edge_index, eps0, W1_0, b1_0, W2_0, b2_0, eps1, W1_1, b1_1, W2_1, b2_1):
    src = edge_index[0]
    dst = edge_index[1]
    zeros = jnp.zeros((ROWS_PER_TILE, D), jnp.float32)
    h = _gin_layer(x, src, dst, zeros, eps0, W1_0, b1_0, W2_0, b2_0)
    h = _gin_layer(h, src, dst, zeros, eps1, W1_1, b1_1, W2_1, b2_1)
    return h


# speedup vs baseline: 13.4278x; 2.1090x over previous
import functools

import jax
import jax.numpy as jnp
from jax import lax
from jax.experimental import pallas as pl
from jax.experimental.pallas import tpu as pltpu
from jax.experimental.pallas import tpu_sc as plsc

N_NODES = 10000
N_EDGES = 320000
D = 128

CHUNK = 128
NC = 2
NS = 16
NW = NC * NS
EPW = N_EDGES // NW
NCH = EPW // CHUNK
TAIL = EPW - NCH * CHUNK
N_PAD = 10112
ROWS_PER_TILE = N_PAD // NS


@functools.partial(
    pl.kernel,
    out_type=jax.ShapeDtypeStruct((NC, N_PAD, D), jnp.float32),
    mesh=plsc.VectorSubcoreMesh(core_axis_name="c", subcore_axis_name="s"),
    scratch_types=[
        pltpu.VMEM((EPW,), jnp.int32),
        pltpu.VMEM((CHUNK,), jnp.int32),
        pltpu.VMEM((CHUNK,), jnp.int32),
        pltpu.VMEM((TAIL,), jnp.int32),
        pltpu.VMEM((CHUNK, D), jnp.float32),
        pltpu.VMEM((CHUNK, D), jnp.float32),
        pltpu.VMEM_SHARED((N_PAD, D), jnp.float32),
        pltpu.SemaphoreType.DMA,
        pltpu.SemaphoreType.DMA,
        pltpu.SemaphoreType.DMA,
        pltpu.SemaphoreType.DMA,
    ],
)
def _sc_aggregate(src_hbm, dst_hbm, h_hbm, out_hbm,
                  src_v, dst0, dst1, dstt, rows0, rows1, acc_sh,
                  gsem0, gsem1, dsem0, dsem1):
    cid = lax.axis_index("c")
    sid = lax.axis_index("s")
    wid = sid * NC + cid
    ebase = wid * EPW

    zvec = jnp.zeros((16,), jnp.float32)

    def zfill(r, carry):
        for q in range(8):
            rows0[r, pl.ds(q * 16, 16)] = zvec
        return carry

    lax.fori_loop(0, CHUNK, zfill, 0)

    def zcopy(k, carry):
        pltpu.sync_copy(rows0, acc_sh.at[pl.ds(sid * ROWS_PER_TILE + k * CHUNK, CHUNK)])
        return carry

    lax.fori_loop(0, 4, zcopy, 0)
    pltpu.sync_copy(
        rows0.at[pl.ds(0, ROWS_PER_TILE - 4 * CHUNK)],
        acc_sh.at[pl.ds(sid * ROWS_PER_TILE + 4 * CHUNK, ROWS_PER_TILE - 4 * CHUNK)],
    )

    pltpu.sync_copy(src_hbm.at[pl.ds(ebase, EPW)], src_v)
    plsc.subcore_barrier()

    dsts = (dst0, dst1)
    dsems = (dsem0, dsem1)
    rows = (rows0, rows1)
    gsems = (gsem0, gsem1)

    def fire_dst(c, b):
        pltpu.async_copy(dst_hbm.at[pl.ds(ebase + c * CHUNK, CHUNK)], dsts[b], dsems[b])

    def fire_gather(c, b):
        pltpu.async_copy(h_hbm.at[src_v.at[pl.ds(c * CHUNK, CHUNK)]], rows[b], gsems[b])

    def wait_dst(c, b):
        pltpu.make_async_copy(
            dst_hbm.at[pl.ds(ebase + c * CHUNK, CHUNK)], dsts[b], dsems[b]).wait()

    def wait_gather(c, b):
        pltpu.make_async_copy(
            h_hbm.at[src_v.at[pl.ds(c * CHUNK, CHUNK)]], rows[b], gsems[b]).wait()

    fire_dst(0, 0)
    fire_dst(1, 1)
    fire_gather(0, 0)

    def body(j, carry):
        c0 = 2 * j
        c1 = 2 * j + 1
        fire_gather(c1, 1)
        wait_gather(c0, 0)
        wait_dst(c0, 0)
        pltpu.sync_copy(rows0, acc_sh.at[dst0], add=True)

        @pl.when(c1 + 1 < NCH)
        def _():
            fire_dst(c0 + 2, 0)
            fire_gather(c1 + 1, 0)

        wait_gather(c1, 1)
        wait_dst(c1, 1)
        pltpu.sync_copy(rows1, acc_sh.at[dst1], add=True)

        @pl.when(c1 + 2 < NCH)
        def _():
            fire_dst(c1 + 2, 1)

        return carry

    lax.fori_loop(0, NCH // 2, body, 0)

    pltpu.sync_copy(dst_hbm.at[pl.ds(ebase + NCH * CHUNK, TAIL)], dstt)
    pltpu.async_copy(
        h_hbm.at[src_v.at[pl.ds(NCH * CHUNK, TAIL)]], rows0.at[pl.ds(0, TAIL)], gsem0
    ).wait()
    pltpu.sync_copy(rows0.at[pl.ds(0, TAIL)], acc_sh.at[dstt], add=True)

    plsc.subcore_barrier()

    pltpu.sync_copy(
        acc_sh.at[pl.ds(sid * ROWS_PER_TILE, ROWS_PER_TILE)],
        out_hbm.at[cid, pl.ds(sid * ROWS_PER_TILE, ROWS_PER_TILE)],
    )


BLK = 2000


def _mlp_body(scale_ref, h_ref, p_ref, w1_ref, b1_ref, w2_ref, b2_ref, o_ref):
    scale = scale_ref[0]
    z = h_ref[...] * scale + p_ref[0] + p_ref[1]
    z = jnp.dot(z, w1_ref[...], preferred_element_type=jnp.float32) + b1_ref[...]
    z = jnp.maximum(z, 0.0)
    z = jnp.dot(z, w2_ref[...], preferred_element_type=jnp.float32) + b2_ref[...]
    o_ref[...] = jnp.maximum(z, 0.0)


_tc_mlp = pl.pallas_call(
    _mlp_body,
    grid=(N_NODES // BLK,),
    in_specs=[
        pl.BlockSpec(memory_space=pltpu.SMEM),
        pl.BlockSpec((BLK, D), lambda i: (i, 0)),
        pl.BlockSpec((NC, BLK, D), lambda i: (0, i, 0)),
        pl.BlockSpec((D, D), lambda i: (0, 0)),
        pl.BlockSpec((1, D), lambda i: (0, 0)),
        pl.BlockSpec((D, D), lambda i: (0, 0)),
        pl.BlockSpec((1, D), lambda i: (0, 0)),
    ],
    out_specs=pl.BlockSpec((BLK, D), lambda i: (i, 0)),
    out_shape=jax.ShapeDtypeStruct((N_NODES, D), jnp.float32),
)


def _gin_layer(h, src, dst, eps, W1, b1, W2, b2):
    parts = _sc_aggregate(src, dst, h)
    scale = (1.0 + eps).reshape((1,)).astype(jnp.float32)
    return _tc_mlp(scale, h, parts, W1, b1.reshape(1, D), W2, b2.reshape(1, D))


def kernel(x, edge_index, eps0, W1_0, b1_0, W2_0, b2_0, eps1, W1_1, b1_1, W2_1, b2_1):
    src = edge_index[0]
    dst = edge_index[1]
    h = _gin_layer(x, src, dst, eps0, W1_0, b1_0, W2_0, b2_0)
    h = _gin_layer(h, src, dst, eps1, W1_1, b1_1, W2_1, b2_1)
    return h


# speedup vs baseline: 15.1126x; 1.1255x over previous
import functools

import jax
import jax.numpy as jnp
from jax import lax
from jax.experimental import pallas as pl
from jax.experimental.pallas import tpu as pltpu
from jax.experimental.pallas import tpu_sc as plsc

N_NODES = 10000
N_EDGES = 320000
D = 128

CHUNK = 64
NC = 2
NS = 16
NW = NC * NS
EPW = N_EDGES // NW
NCH = EPW // CHUNK
TAIL = EPW - NCH * CHUNK
NBUF = 4
NQ = NCH // NBUF
N_PAD = 10112
ROWS_PER_TILE = N_PAD // NS


@functools.partial(
    pl.kernel,
    out_type=jax.ShapeDtypeStruct((NC, N_PAD, D), jnp.float32),
    mesh=plsc.VectorSubcoreMesh(core_axis_name="c", subcore_axis_name="s"),
    scratch_types=[
        pltpu.VMEM((EPW,), jnp.int32),
        pltpu.VMEM((CHUNK,), jnp.int32),
        pltpu.VMEM((CHUNK,), jnp.int32),
        pltpu.VMEM((CHUNK,), jnp.int32),
        pltpu.VMEM((CHUNK,), jnp.int32),
        pltpu.VMEM((TAIL,), jnp.int32),
        pltpu.VMEM((CHUNK, D), jnp.float32),
        pltpu.VMEM((CHUNK, D), jnp.float32),
        pltpu.VMEM((CHUNK, D), jnp.float32),
        pltpu.VMEM((CHUNK, D), jnp.float32),
        pltpu.VMEM_SHARED((N_PAD, D), jnp.float32),
        pltpu.SemaphoreType.DMA,
        pltpu.SemaphoreType.DMA,
        pltpu.SemaphoreType.DMA,
        pltpu.SemaphoreType.DMA,
        pltpu.SemaphoreType.DMA,
        pltpu.SemaphoreType.DMA,
        pltpu.SemaphoreType.DMA,
        pltpu.SemaphoreType.DMA,
        pltpu.SemaphoreType.DMA,
        pltpu.SemaphoreType.DMA,
        pltpu.SemaphoreType.DMA,
        pltpu.SemaphoreType.DMA,
    ],
)
def _sc_aggregate(src_hbm, dst_hbm, h_hbm, out_hbm,
                  src_v, dstA, dstB, dstC, dstD, dstt,
                  rows0, rows1, rows2, rows3, acc_sh,
                  g0, g1, g2, g3, d0, d1, d2, d3, s0, s1, s2, s3):
    cid = lax.axis_index("c")
    sid = lax.axis_index("s")
    wid = sid * NC + cid
    ebase = wid * EPW

    rows = (rows0, rows1, rows2, rows3)
    dsts = (dstA, dstB, dstC, dstD)
    gsem = (g0, g1, g2, g3)
    dsem = (d0, d1, d2, d3)
    ssem = (s0, s1, s2, s3)

    zvec = jnp.zeros((16,), jnp.float32)

    def zfill(r, carry):
        for q in range(8):
            rows0[r, pl.ds(q * 16, 16)] = zvec
        return carry

    lax.fori_loop(0, CHUNK, zfill, 0)

    def zcopy(k, carry):
        pltpu.sync_copy(rows0, acc_sh.at[pl.ds(sid * ROWS_PER_TILE + k * CHUNK, CHUNK)])
        return carry

    lax.fori_loop(0, 4, zcopy, 0)
    pltpu.sync_copy(
        rows0.at[pl.ds(0, ROWS_PER_TILE - 4 * CHUNK)],
        acc_sh.at[pl.ds(sid * ROWS_PER_TILE + 4 * CHUNK, ROWS_PER_TILE - 4 * CHUNK)],
    )

    pltpu.sync_copy(src_hbm.at[pl.ds(ebase, EPW)], src_v)
    plsc.subcore_barrier()

    def fire_gather(c, b):
        pltpu.async_copy(h_hbm.at[src_v.at[pl.ds(c * CHUNK, CHUNK)]], rows[b], gsem[b])

    def wait_gather(b):
        pltpu.make_async_copy(
            h_hbm.at[src_v.at[pl.ds(0, CHUNK)]], rows[b], gsem[b]).wait()

    def fire_dst(c, b):
        pltpu.async_copy(dst_hbm.at[pl.ds(ebase + c * CHUNK, CHUNK)], dsts[b], dsem[b])

    def wait_dst(b):
        pltpu.make_async_copy(
            dst_hbm.at[pl.ds(ebase, CHUNK)], dsts[b], dsem[b]).wait()

    def fire_scatter(c, b):
        pltpu.async_copy(rows[b], acc_sh.at[dsts[b]], ssem[b], add=True)

    def wait_scatter(b):
        pltpu.make_async_copy(rows[b], acc_sh.at[dsts[b]], ssem[b]).wait()

    for b in range(3):
        fire_dst(b, b)
        fire_gather(b, b)

    def body(j, carry):
        for p in range(NBUF):
            c = NBUF * j + p
            pn = (p + 3) % NBUF

            @pl.when(c + 3 < NCH)
            def _():
                @pl.when(c >= 1)
                def _():
                    wait_scatter(pn)

                fire_dst(c + 3, pn)
                fire_gather(c + 3, pn)

            wait_gather(p)
            wait_dst(p)
            fire_scatter(c, p)
        return carry

    lax.fori_loop(0, NQ, body, 0)

    for p in range(NCH - NBUF * NQ):
        wait_gather(p)
        wait_dst(p)
        fire_scatter(NBUF * NQ + p, p)

    for p in range(NBUF):
        wait_scatter(p)

    pltpu.sync_copy(dst_hbm.at[pl.ds(ebase + NCH * CHUNK, TAIL)], dstt)
    pltpu.async_copy(
        h_hbm.at[src_v.at[pl.ds(NCH * CHUNK, TAIL)]], rows0.at[pl.ds(0, TAIL)], g0
    ).wait()
    pltpu.sync_copy(rows0.at[pl.ds(0, TAIL)], acc_sh.at[dstt], add=True)

    plsc.subcore_barrier()

    pltpu.sync_copy(
        acc_sh.at[pl.ds(sid * ROWS_PER_TILE, ROWS_PER_TILE)],
        out_hbm.at[cid, pl.ds(sid * ROWS_PER_TILE, ROWS_PER_TILE)],
    )


BLK = 2000


def _mlp_body(scale_ref, h_ref, p_ref, w1_ref, b1_ref, w2_ref, b2_ref, o_ref):
    scale = scale_ref[0]
    z = h_ref[...] * scale + p_ref[0] + p_ref[1]
    z = jnp.dot(z, w1_ref[...], preferred_element_type=jnp.float32) + b1_ref[...]
    z = jnp.maximum(z, 0.0)
    z = jnp.dot(z, w2_ref[...], preferred_element_type=jnp.float32) + b2_ref[...]
    o_ref[...] = jnp.maximum(z, 0.0)


_tc_mlp = pl.pallas_call(
    _mlp_body,
    grid=(N_NODES // BLK,),
    in_specs=[
        pl.BlockSpec(memory_space=pltpu.SMEM),
        pl.BlockSpec((BLK, D), lambda i: (i, 0)),
        pl.BlockSpec((NC, BLK, D), lambda i: (0, i, 0)),
        pl.BlockSpec((D, D), lambda i: (0, 0)),
        pl.BlockSpec((1, D), lambda i: (0, 0)),
        pl.BlockSpec((D, D), lambda i: (0, 0)),
        pl.BlockSpec((1, D), lambda i: (0, 0)),
    ],
    out_specs=pl.BlockSpec((BLK, D), lambda i: (i, 0)),
    out_shape=jax.ShapeDtypeStruct((N_NODES, D), jnp.float32),
)


def _gin_layer(h, src, dst, eps, W1, b1, W2, b2):
    parts = _sc_aggregate(src, dst, h)
    scale = (1.0 + eps).reshape((1,)).astype(jnp.float32)
    return _tc_mlp(scale, h, parts, W1, b1.reshape(1, D), W2, b2.reshape(1, D))


def kernel(x, edge_index, eps0, W1_0, b1_0, W2_0, b2_0, eps1, W1_1, b1_1, W2_1, b2_1):
    src = edge_index[0]
    dst = edge_index[1]
    h = _gin_layer(x, src, dst, eps0, W1_0, b1_0, W2_0, b2_0)
    h = _gin_layer(h, src, dst, eps1, W1_1, b1_1, W2_1, b2_1)
    return h
